# baseline (speedup 1.0000x reference)
import functools

import jax
import jax.numpy as jnp
from jax import lax
from jax.experimental import pallas as pl
from jax.experimental.pallas import tpu as pltpu
from jax.experimental.pallas import tpu_sc as plsc

_N = 100000
_D = 128
_BLK = 1000

_NW = 32
_NB = 8
_ROWS = 12544
_NPAD = _NB * _ROWS
_DEGW = 101376
_EP = 100352
_EWK = _EP // _NW
_SLOT = _EWK + 64
_FIRE = 64
_SENT = 0x3FFFFFFF
_SHIFT = 17


def _mesh():
    return plsc.VectorSubcoreMesh(core_axis_name="c", subcore_axis_name="s")


def _bin_body(src_ref, dst_ref, bins_ref, cnts_ref, deg_ref,
              esrc, edst, ebuf, didx, ones64, cntv, zbuf, deg_sh):
    c = lax.axis_index("c")
    s = lax.axis_index("s")
    w = c * 16 + s
    iota16 = lax.iota(jnp.int32, 16)
    zero16 = jnp.zeros((16,), jnp.float32)
    one16 = jnp.ones((16,), jnp.float32)

    pltpu.sync_copy(src_ref.at[pl.ds(w * _EWK, _EWK)], esrc)
    pltpu.sync_copy(dst_ref.at[pl.ds(w * _EWK, _EWK)], edst)

    def _zb(r, carry):
        zbuf[pl.ds(r * 16, 16)] = zero16
        return carry
    lax.fori_loop(0, 198, _zb, 0)
    for t in range(4):
        ones64[pl.ds(t * 16, 16)] = one16
    cntv[pl.ds(0, 16)] = jnp.zeros((16,), jnp.int32)

    pltpu.sync_copy(zbuf, deg_sh.at[pl.ds(s * 6336, 3168)])
    pltpu.sync_copy(zbuf, deg_sh.at[pl.ds(s * 6336 + 3168, 3168)])
    plsc.subcore_barrier()

    for blk in range(_NB):
        lo = blk * _ROWS

        def _scan(k, cnt):
            base = k * 16
            dvec = edst[pl.ds(base, 16)]
            svec = esrc[pl.ds(base, 16)]
            rel = dvec - lo
            m = (rel >= 0) & (rel < _ROWS)
            mi = m.astype(jnp.int32)
            pos = cnt + plsc.cumsum(mi) - 1
            pk = svec | lax.shift_left(rel, _SHIFT)
            plsc.store_scatter(ebuf, [pos], pk, mask=m)
            return cnt + jnp.sum(mi)
        cnt = lax.fori_loop(0, _EWK // 16, _scan, jnp.int32(0))

        pad = jnp.full((16,), _ROWS << _SHIFT, jnp.int32)
        for t in range(4):
            plsc.store_scatter(ebuf, [cnt + t * 16 + iota16], pad)
        nf = (cnt + (_FIRE - 1)) // _FIRE
        plsc.store_scatter(cntv, [jnp.full((16,), blk, jnp.int32)],
                           jnp.full((16,), nf, jnp.int32), mask=iota16 == 0)

        def _degf(f, carry):
            for t in range(4):
                pk = ebuf[pl.ds(f * _FIRE + t * 16, 16)]
                rel = lax.shift_right_logical(pk, _SHIFT)
                didx[pl.ds(t * 16, 16)] = jnp.where(rel >= _ROWS, _NPAD, rel + lo)
            pltpu.sync_copy(ones64, deg_sh.at[didx], add=True)
            return carry
        lax.fori_loop(0, nf, _degf, 0)

        pltpu.sync_copy(ebuf, bins_ref.at[pl.ds((blk * _NW + w) * _SLOT, _SLOT)])

    pltpu.sync_copy(cntv, cnts_ref.at[pl.ds(w * 16, 16)])
    plsc.subcore_barrier()
    for h in range(2):
        pltpu.sync_copy(deg_sh.at[pl.ds(s * 6336 + h * 3168, 3168)], zbuf)
        pltpu.sync_copy(zbuf, deg_ref.at[pl.ds(c * _DEGW + s * 6336 + h * 3168, 3168)])


def _bin(src, dst):
    kern = pl.kernel(
        _bin_body,
        out_type=(
            jax.ShapeDtypeStruct((_NB * _NW * _SLOT,), jnp.int32),
            jax.ShapeDtypeStruct((_NW * 16,), jnp.int32),
            jax.ShapeDtypeStruct((2 * _DEGW,), jnp.float32),
        ),
        mesh=_mesh(),
        compiler_params=pltpu.CompilerParams(needs_layout_passes=False),
        scratch_types=[
            pltpu.VMEM((_EWK,), jnp.int32),
            pltpu.VMEM((_EWK,), jnp.int32),
            pltpu.VMEM((_SLOT,), jnp.int32),
            pltpu.VMEM((_FIRE,), jnp.int32),
            pltpu.VMEM((_FIRE,), jnp.float32),
            pltpu.VMEM((16,), jnp.int32),
            pltpu.VMEM((3168,), jnp.float32),
            pltpu.VMEM_SHARED((_DEGW,), jnp.float32),
        ],
    )
    return kern(src, dst)


def _smooth_body(width, g_ref, bins_ref, cnts_ref, s_ref,
                 ebuf, cntb, gidx0, gidx1, sidx0, sidx1, rows0, rows1,
                 zrows, acc, semg, semz):
    c = lax.axis_index("c")
    s = lax.axis_index("s")
    nchunk = width // 16
    zero16 = jnp.zeros((16,), jnp.float32)
    iota16 = lax.iota(jnp.int32, 16)

    pltpu.sync_copy(cnts_ref.at[pl.ds(s * 16, 16)], cntb.at[pl.ds(0, 16)])
    pltpu.sync_copy(cnts_ref.at[pl.ds((16 + s) * 16, 16)], cntb.at[pl.ds(16, 16)])
    cntvecs = [cntb[pl.ds(0, 16)], cntb[pl.ds(16, 16)]]

    def _zb(r, carry):
        for t in range(nchunk):
            zrows[r, pl.ds(t * 16, 16)] = zero16
        return carry
    lax.fori_loop(0, 16, _zb, 0)

    rows_per_sub = _ROWS // 16

    def _prep(f, gi, si):
        for t in range(4):
            pk = ebuf[pl.ds(f * _FIRE + t * 16, 16)]
            gi[pl.ds(t * 16, 16)] = pk & ((1 << _SHIFT) - 1)
            si[pl.ds(t * 16, 16)] = lax.shift_right_logical(pk, _SHIFT)

    for t in range(_NB // 2):
        b = c * (_NB // 2) + t
        lo = b * _ROWS

        descs = [pltpu.async_copy(
            zrows, acc.at[pl.ds(s * rows_per_sub + z * 16, 16)], semz)
            for z in range(rows_per_sub // 16)]
        for d in descs:
            d.wait()
        plsc.subcore_barrier()

        for h in range(2):
            w2 = h * 16 + s
            pltpu.sync_copy(bins_ref.at[pl.ds((b * _NW + w2) * _SLOT, _SLOT)], ebuf)
            nf = jnp.sum(jnp.where(iota16 == b, cntvecs[h], 0))

            @pl.when(nf > 0)
            def _():
                _prep(0, gidx0, sidx0)
                pltpu.async_copy(g_ref.at[gidx0], rows0, semg)

            def _body(g, carry):
                f1 = g * 2 + 1
                pltpu.make_async_copy(g_ref.at[gidx0], rows0, semg).wait()

                @pl.when(f1 < nf)
                def _():
                    _prep(f1, gidx1, sidx1)
                    pltpu.async_copy(g_ref.at[gidx1], rows1, semg)
                pltpu.sync_copy(rows0, acc.at[sidx0], add=True)

                @pl.when(f1 < nf)
                def _():
                    pltpu.make_async_copy(g_ref.at[gidx1], rows1, semg).wait()

                    @pl.when(f1 + 1 < nf)
                    def _():
                        _prep(f1 + 1, gidx0, sidx0)
                        pltpu.async_copy(g_ref.at[gidx0], rows0, semg)
                    pltpu.sync_copy(rows1, acc.at[sidx1], add=True)
                return carry
            lax.fori_loop(0, (nf + 1) // 2, _body, 0)
        plsc.subcore_barrier()

        descs = [pltpu.async_copy(
            acc.at[pl.ds(s * rows_per_sub + z * 112, 112)],
            s_ref.at[pl.ds(lo + s * rows_per_sub + z * 112, 112)], semz)
            for z in range(7)]
        for d in descs:
            d.wait()
        plsc.subcore_barrier()


def _smooth(G, bins, cnts):
    width = G.shape[1]
    kern = pl.kernel(
        functools.partial(_smooth_body, width),
        out_type=jax.ShapeDtypeStruct((_NPAD, width), jnp.float32),
        mesh=_mesh(),
        compiler_params=pltpu.CompilerParams(needs_layout_passes=False),
        scratch_types=[
            pltpu.VMEM((_SLOT,), jnp.int32),
            pltpu.VMEM((32,), jnp.int32),
            pltpu.VMEM((_FIRE,), jnp.int32),
            pltpu.VMEM((_FIRE,), jnp.int32),
            pltpu.VMEM((_FIRE,), jnp.int32),
            pltpu.VMEM((_FIRE,), jnp.int32),
            pltpu.VMEM((_FIRE, width), jnp.float32),
            pltpu.VMEM((_FIRE, width), jnp.float32),
            pltpu.VMEM((16, width), jnp.float32),
            pltpu.VMEM_SHARED((_ROWS + 8, width), jnp.float32),
            pltpu.SemaphoreType.DMA,
            pltpu.SemaphoreType.DMA,
        ],
    )
    return kern(G, bins, cnts)


def _l0_body(x_ref, w_ref, b_ref, deg_ref, o_ref):
    dinv = jax.lax.rsqrt(deg_ref[...])
    h = jnp.dot(x_ref[...], w_ref[...], preferred_element_type=jnp.float32)
    o_ref[...] = (h + b_ref[...]) * dinv


def _mid_body(s_ref, g_ref, deg_ref, w_ref, b_ref, o_ref):
    dinv = jax.lax.rsqrt(deg_ref[...])
    h_in = jnp.maximum(dinv * (s_ref[...] + g_ref[...]), 0.0)
    h = jnp.dot(h_in, w_ref[...], preferred_element_type=jnp.float32)
    o_ref[...] = (h + b_ref[...]) * dinv


def _fin_body(s_ref, g_ref, deg_ref, o_ref):
    dinv = jax.lax.rsqrt(deg_ref[...])
    o_ref[...] = dinv * (s_ref[...] + g_ref[...])


def _row_spec(width):
    return pl.BlockSpec((_BLK, width), lambda i: (i, 0))


def _full_spec(shape):
    return pl.BlockSpec(shape, lambda i: (0, 0))


def _layer0(X, W, b, deg):
    return pl.pallas_call(
        _l0_body,
        grid=(_N // _BLK,),
        in_specs=[
            _row_spec(_D),
            _full_spec(W.shape),
            _full_spec((1, W.shape[1])),
            _row_spec(1),
        ],
        out_specs=_row_spec(W.shape[1]),
        out_shape=jax.ShapeDtypeStruct((_N, W.shape[1]), jnp.float32),
    )(X, W, b.reshape(1, -1), deg)


def _layer_mid(S, G, deg, W, b):
    return pl.pallas_call(
        _mid_body,
        grid=(_N // _BLK,),
        in_specs=[
            _row_spec(_D),
            _row_spec(_D),
            _row_spec(1),
            _full_spec(W.shape),
            _full_spec((1, W.shape[1])),
        ],
        out_specs=_row_spec(W.shape[1]),
        out_shape=jax.ShapeDtypeStruct((_N, W.shape[1]), jnp.float32),
    )(S, G, deg, W, b.reshape(1, -1))


def _layer_fin(S, G, deg):
    width = G.shape[1]
    return pl.pallas_call(
        _fin_body,
        grid=(_N // _BLK,),
        in_specs=[_row_spec(width), _row_spec(width), _row_spec(1)],
        out_specs=_row_spec(width),
        out_shape=jax.ShapeDtypeStruct((_N, width), jnp.float32),
    )(S, G, deg)


def kernel(X, hyperedges, W0, b0, W1, b1, W2, b2):
    he = hyperedges.astype(jnp.int32)
    E, K = he.shape

    Xe = X[he]
    sq = jnp.sum(Xe * Xe, axis=-1)
    gram = jnp.einsum('ekd,emd->ekm', Xe, Xe)
    dist = sq[:, :, None] + sq[:, None, :] - 2.0 * gram
    flat = jnp.argmax(dist.reshape(E, K * K), axis=1)
    i = flat // K
    j = flat % K
    ar = jnp.arange(E)
    u = he[ar, i]
    v = he[ar, j]
    src = jnp.concatenate([u, v])
    dst = jnp.concatenate([v, u])

    npad = _EP - src.shape[0]
    src_p = jnp.concatenate([src, jnp.zeros((npad,), jnp.int32)])
    dst_p = jnp.concatenate([dst, jnp.full((npad,), _SENT, jnp.int32)])

    bins, cnts, degp = _bin(src_p, dst_p)
    deg = (degp[:_N] + degp[_DEGW:_DEGW + _N] + 1.0).reshape(_N, 1)

    W2p = jnp.pad(W2, ((0, 0), (0, 88)))
    b2p = jnp.pad(b2, (0, 88))

    G0 = _layer0(X, W0, b0, deg)
    S0 = _smooth(G0, bins, cnts)[:_N]
    G1 = _layer_mid(S0, G0, deg, W1, b1)
    S1 = _smooth(G1, bins, cnts)[:_N]
    G2 = _layer_mid(S1, G1, deg, W2p, b2p)
    S2 = _smooth(G2, bins, cnts)[:_N]
    return _layer_fin(S2, G2, deg)[:, :40]
